# baseline (speedup 1.0000x reference)
import functools

import jax
import jax.numpy as jnp
from jax.experimental import pallas as pl
from jax.experimental.pallas import tpu as pltpu

_HEAD = 16


def _groupA_body(x_ref, w1_ref, b1_ref, w2_ref, b2_ref, o_ref, pf, xs, sem, *, h, w, wp, ns):
    m = h * wp
    ms = m // ns
    jc = jax.lax.broadcasted_iota(jnp.int32, (ms, 64), 0) & (wp - 1)
    pf[...] = jnp.zeros(pf.shape, pf.dtype)
    for s in range(ns):
        soff = s * ms
        cp = pltpu.make_async_copy(x_ref.at[pl.ds(soff, ms + 2 * wp), :], xs, sem)
        cp.start()
        cp.wait()
        acc = None
        for dh in range(3):
            p = jnp.dot(xs[pl.ds(dh * wp, ms), :], w1_ref[dh],
                        preferred_element_type=jnp.float32)
            acc = p if acc is None else acc + p
        act = jnp.maximum(acc + b1_ref[...], 0.0)
        act = jnp.where(jc < w, act, 0.0)
        pf[pl.ds(_HEAD + wp + soff, ms), :] = act.astype(jnp.bfloat16)
    for s in range(ns):
        soff = s * ms
        acc = None
        for dh in range(3):
            base = _HEAD - 1 + dh * wp + soff
            for dw in range(3):
                p = jnp.dot(pf[pl.ds(base + dw, ms), :], w2_ref[dh * 3 + dw],
                            preferred_element_type=jnp.float32)
                acc = p if acc is None else acc + p
        act = jnp.maximum(acc + b2_ref[...], 0.0)
        o_ref[pl.ds(soff, ms), :] = act.astype(jnp.bfloat16)


def _groupA(x, w1, b1, w2, b2, wp, ns):
    h, w, _ = x.shape
    m = h * wp
    xp = jnp.pad(x, ((1, 1), (1, 1), (0, 0)))
    xcat = jnp.concatenate([xp[:, d:d + w, :] for d in range(3)], axis=2)
    xcat = jnp.pad(xcat, ((0, 0), (0, wp - w), (0, 0))).reshape((h + 2) * wp, 9)
    return pl.pallas_call(
        functools.partial(_groupA_body, h=h, w=w, wp=wp, ns=ns),
        in_specs=[pl.BlockSpec(memory_space=pltpu.MemorySpace.HBM)] + [pl.BlockSpec()] * 4,
        out_shape=jax.ShapeDtypeStruct((m, 64), jnp.bfloat16),
        scratch_shapes=[pltpu.VMEM((m + 2 * wp + 2 * _HEAD, 64), jnp.bfloat16),
                        pltpu.VMEM((m // ns + 2 * wp, 9), jnp.bfloat16),
                        pltpu.SemaphoreType.DMA],
    )(xcat, w1.astype(jnp.bfloat16), b1.reshape(1, 64),
      w2.astype(jnp.bfloat16), b2.reshape(1, 64))


def _group_body(*refs, nv, nl, h, w, wp, edge, ns, out_dtype):
    m = h * wp
    ms = m // ns
    views, wbs = refs[:nv], refs[nv:nv + 2 * nl]
    o_ref = refs[nv + 2 * nl]
    pfs = refs[nv + 1 + 2 * nl:]
    def jc(c):
        return jax.lax.broadcasted_iota(jnp.int32, (ms, c), 0) & (wp - 1)
    c0 = pfs[0].shape[1]
    if not edge:
        ztop = jnp.zeros((_HEAD + wp, c0), pfs[0].dtype)
        pfs[0][pl.ds(0, _HEAD + wp), :] = ztop
        pfs[0][pl.ds(_HEAD + (h + 1) * wp, wp + _HEAD), :] = jnp.zeros(
            (wp + _HEAD, c0), pfs[0].dtype)
    vc = views[0][...]
    cur = jnp.maximum(jnp.maximum(vc[:, :c0], vc[:, c0:2 * c0]),
                      jnp.maximum(vc[:, 2 * c0:3 * c0], vc[:, 3 * c0:]))
    pfs[0][pl.ds(_HEAD + wp, m), :] = cur.astype(jnp.bfloat16)
    for l in range(nl):
        pf = pfs[l]
        if edge:
            pf[pl.ds(_HEAD, wp), :] = pf[pl.ds(_HEAD + wp, wp), :]
            pf[pl.ds(_HEAD + (h + 1) * wp, wp), :] = pf[pl.ds(_HEAD + h * wp, wp), :]
        elif l + 1 < nl:
            pfs[l + 1][...] = jnp.zeros(pfs[l + 1].shape, pfs[l + 1].dtype)
        w_ref, b_ref = wbs[2 * l], wbs[2 * l + 1]
        cin = pf.shape[1]
        jci = jc(cin) if edge else None
        for s in range(ns):
            soff = s * ms
            acc = None
            for dh in range(3):
                base = _HEAD - 1 + dh * wp + soff
                s1 = pf[pl.ds(base + 1, ms), :]
                s0 = pf[pl.ds(base, ms), :]
                s2 = pf[pl.ds(base + 2, ms), :]
                if edge:
                    s0 = jnp.where(jci == 0, s1, s0)
                    s2 = jnp.where(jci == w - 1, s1, s2)
                for dw, sv in ((0, s0), (1, s1), (2, s2)):
                    p = jnp.dot(sv, w_ref[dh * 3 + dw], preferred_element_type=jnp.float32)
                    acc = p if acc is None else acc + p
            act = jnp.maximum(acc + b_ref[...], 0.0)
            if l == nl - 1:
                o_ref[pl.ds(soff, ms), :] = act.astype(out_dtype)
            else:
                if not edge:
                    act = jnp.where(jc(act.shape[1]) < w, act, 0.0)
                pfs[l + 1][pl.ds(_HEAD + wp + soff, ms), :] = act.astype(jnp.bfloat16)


def _group(views, wbs, h, w, wp, edge, ns=1, out_dtype=jnp.bfloat16):
    m = h * wp
    nl = len(wbs)
    cout = wbs[-1][0].shape[2]
    cins = [wb[0].shape[1] for wb in wbs]
    args = []
    for wt, b in wbs:
        args += [wt.astype(jnp.bfloat16), b.reshape(1, -1)]
    return pl.pallas_call(
        functools.partial(_group_body, nv=len(views), nl=nl, h=h, w=w, wp=wp,
                          edge=edge, ns=ns, out_dtype=out_dtype),
        out_shape=jax.ShapeDtypeStruct((m, cout), out_dtype),
        scratch_shapes=[pltpu.VMEM((m + 2 * wp + 2 * _HEAD, c), jnp.bfloat16)
                        for c in cins],
    )(*views, *args)


def _pool_views(a, h, w, wp_in, wp_out):
    c = a.shape[1]
    x3 = a.reshape(h, wp_in, c)[:, :w, :]
    h2, w2 = h // 2, w // 2
    vs = [jnp.pad(x3[i::2, j::2, :], ((0, 0), (0, wp_out - w2), (0, 0)))
          for i in (0, 1) for j in (0, 1)]
    return [jnp.concatenate(vs, axis=2).reshape(h2 * wp_out, 4 * c)]


def _t9(cw):
    return cw.transpose(2, 3, 1, 0).reshape(9, cw.shape[1], cw.shape[0])


def kernel(batch, pooling_mask, c1_w, c1_b, c2_w, c2_b, c3_w, c3_b, c4_w, c4_b,
           W5, b5, W6, b6, W7, b7, W8, b8, W9, b9, W10, b10):
    x = batch[0].transpose(1, 2, 0).astype(jnp.bfloat16)
    a = _groupA(x, _t9(c1_w).reshape(3, 9, 64), c1_b, _t9(c2_w), c2_b,
                wp=256, ns=8)
    a = _group(_pool_views(a, 224, 224, 256, 128),
               [(_t9(c3_w), c3_b), (_t9(c4_w), c4_b)],
               h=112, w=112, wp=128, edge=False)
    a = _group(_pool_views(a, 112, 112, 128, 64),
               [(W5.reshape(9, 128, 256), b5),
                (W6.reshape(9, 256, 256), b6),
                (W7.reshape(9, 256, 256), b7)],
               h=56, w=56, wp=64, edge=True)
    a = _group(_pool_views(a, 56, 56, 64, 32),
               [(W8.reshape(9, 256, 512), b8),
                (W9.reshape(9, 512, 512), b9),
                (W10.reshape(9, 512, 512), b10)],
               h=28, w=28, wp=32, edge=True, out_dtype=jnp.float32)
    return a.reshape(28, 32, 512)[:, :28, :].transpose(2, 0, 1)[None]


# speedup vs baseline: 1.0348x; 1.0348x over previous
import functools

import jax
import jax.numpy as jnp
from jax.experimental import pallas as pl
from jax.experimental.pallas import tpu as pltpu

_HEAD = 16


def _conv1_body(x_ref, w_ref, b_ref, o_ref, *, m, wp, w, ns):
    ms = m // ns
    cout = o_ref.shape[1]
    jc = jax.lax.broadcasted_iota(jnp.int32, (ms, cout), 0) & (wp - 1)
    for s in range(ns):
        soff = s * ms
        acc = None
        for dh in range(3):
            x = x_ref[pl.ds(dh * wp + soff, ms), :]
            p = jnp.dot(x, w_ref[dh], preferred_element_type=jnp.float32)
            acc = p if acc is None else acc + p
        act = jnp.maximum(acc + b_ref[...], 0.0)
        act = jnp.where(jc < w, act, 0.0)
        o_ref[pl.ds(soff, ms), :] = act.astype(jnp.bfloat16)


def _conv1(x, wt, b, wp, ns):
    h, w, cin = x.shape
    cout = wt.shape[2]
    m = h * wp
    xp = jnp.pad(x, ((1, 1), (1, 1), (0, 0)))
    xcat = jnp.concatenate([xp[:, d:d + w, :] for d in range(3)], axis=2)
    xcat = jnp.pad(xcat, ((0, 0), (0, wp - w), (0, 0))).reshape((h + 2) * wp, 3 * cin)
    return pl.pallas_call(
        functools.partial(_conv1_body, m=m, wp=wp, w=w, ns=ns),
        out_shape=jax.ShapeDtypeStruct((m, cout), jnp.bfloat16),
    )(xcat, wt, b.reshape(1, cout))


def _group_body(*refs, nv, nl, h, w, wp, edge, ns, dma_in, out_dtype):
    m = h * wp
    ms = m // ns
    views, wbs = refs[:nv], refs[nv:nv + 2 * nl]
    o_ref = refs[nv + 2 * nl]
    pfs = refs[nv + 1 + 2 * nl:]
    def jc(c):
        return jax.lax.broadcasted_iota(jnp.int32, (ms, c), 0) & (wp - 1)
    if dma_in:
        sem = pfs[-1]
        pfs = pfs[:-1]
    if not edge:
        pfs[0][...] = jnp.zeros(pfs[0].shape, pfs[0].dtype)
    if dma_in:
        cp = pltpu.make_async_copy(views[0], pfs[0].at[pl.ds(_HEAD + wp, m), :], sem)
        cp.start()
        cp.wait()
    else:
        if nv == 4:
            cur = jnp.maximum(jnp.maximum(views[0][...], views[1][...]),
                              jnp.maximum(views[2][...], views[3][...]))
        else:
            cur = views[0][...]
        pfs[0][pl.ds(_HEAD + wp, m), :] = cur.astype(jnp.bfloat16)
    for l in range(nl):
        pf = pfs[l]
        if edge:
            pf[pl.ds(_HEAD, wp), :] = pf[pl.ds(_HEAD + wp, wp), :]
            pf[pl.ds(_HEAD + (h + 1) * wp, wp), :] = pf[pl.ds(_HEAD + h * wp, wp), :]
        elif l + 1 < nl:
            pfs[l + 1][...] = jnp.zeros(pfs[l + 1].shape, pfs[l + 1].dtype)
        w_ref, b_ref = wbs[2 * l], wbs[2 * l + 1]
        cin = pf.shape[1]
        jci = jc(cin) if edge else None
        for s in range(ns):
            soff = s * ms
            acc = None
            for dh in range(3):
                base = _HEAD - 1 + dh * wp + soff
                s1 = pf[pl.ds(base + 1, ms), :]
                s0 = pf[pl.ds(base, ms), :]
                s2 = pf[pl.ds(base + 2, ms), :]
                if edge:
                    s0 = jnp.where(jci == 0, s1, s0)
                    s2 = jnp.where(jci == w - 1, s1, s2)
                for dw, sv in ((0, s0), (1, s1), (2, s2)):
                    p = jnp.dot(sv, w_ref[dh * 3 + dw], preferred_element_type=jnp.float32)
                    acc = p if acc is None else acc + p
            act = jnp.maximum(acc + b_ref[...], 0.0)
            if l == nl - 1:
                o_ref[pl.ds(soff, ms), :] = act.astype(out_dtype)
            else:
                if not edge:
                    act = jnp.where(jc(act.shape[1]) < w, act, 0.0)
                pfs[l + 1][pl.ds(_HEAD + wp + soff, ms), :] = act.astype(jnp.bfloat16)


def _group(views, wbs, h, w, wp, edge, ns=1, dma_in=False, out_dtype=jnp.bfloat16):
    m = h * wp
    nl = len(wbs)
    cout = wbs[-1][0].shape[2]
    cins = [wb[0].shape[1] for wb in wbs]
    args = []
    for wt, b in wbs:
        args += [wt.astype(jnp.bfloat16), b.reshape(1, -1)]
    scratch = [pltpu.VMEM((m + 2 * wp + 2 * _HEAD, c), jnp.bfloat16) for c in cins]
    in_specs = [pl.BlockSpec(memory_space=pltpu.MemorySpace.HBM) if dma_in else pl.BlockSpec()
                for _ in views] + [pl.BlockSpec() for _ in args]
    if dma_in:
        scratch = scratch + [pltpu.SemaphoreType.DMA]
    return pl.pallas_call(
        functools.partial(_group_body, nv=len(views), nl=nl, h=h, w=w, wp=wp,
                          edge=edge, ns=ns, dma_in=dma_in, out_dtype=out_dtype),
        in_specs=in_specs,
        out_shape=jax.ShapeDtypeStruct((m, cout), out_dtype),
        scratch_shapes=scratch,
    )(*views, *args)


def _pool_views(a, h, w, wp_in, wp_out):
    c = a.shape[1]
    x3 = a.reshape(h, wp_in, c)[:, :w, :]
    h2, w2 = h // 2, w // 2
    return [jnp.pad(x3[i::2, j::2, :], ((0, 0), (0, wp_out - w2), (0, 0)))
            .reshape(h2 * wp_out, c) for i in (0, 1) for j in (0, 1)]


def _t9(cw):
    return cw.transpose(2, 3, 1, 0).reshape(9, cw.shape[1], cw.shape[0])


def kernel(batch, pooling_mask, c1_w, c1_b, c2_w, c2_b, c3_w, c3_b, c4_w, c4_b,
           W5, b5, W6, b6, W7, b7, W8, b8, W9, b9, W10, b10):
    x = batch[0].transpose(1, 2, 0).astype(jnp.bfloat16)
    a = _conv1(x, _t9(c1_w).reshape(3, 9, 64).astype(jnp.bfloat16), c1_b,
               wp=256, ns=2)
    a = _group([a], [(_t9(c2_w), c2_b)], h=224, w=224, wp=256, edge=False, ns=4, dma_in=True)
    a = _group(_pool_views(a, 224, 224, 256, 128),
               [(_t9(c3_w), c3_b), (_t9(c4_w), c4_b)],
               h=112, w=112, wp=128, edge=False)
    a = _group(_pool_views(a, 112, 112, 128, 64),
               [(W5.reshape(9, 128, 256), b5),
                (W6.reshape(9, 256, 256), b6),
                (W7.reshape(9, 256, 256), b7)],
               h=56, w=56, wp=64, edge=True)
    a = _group(_pool_views(a, 56, 56, 64, 32),
               [(W8.reshape(9, 256, 512), b8),
                (W9.reshape(9, 512, 512), b9),
                (W10.reshape(9, 512, 512), b10)],
               h=28, w=28, wp=32, edge=True, out_dtype=jnp.float32)
    return a.reshape(28, 32, 512)[:, :28, :].transpose(2, 0, 1)[None]
